# Initial kernel scaffold; baseline (speedup 1.0000x reference)
import jax
import jax.numpy as jnp
from jax.experimental import pallas as pl


def kernel(x, edge_index, edge_weight, edge_index_sup, edge_weight_sup, mask, A_scaler, truths, eps_x, eps_d, h0, params):
    raise NotImplementedError("write your pallas kernel here")


# speedup vs baseline: 12.2461x; 12.2461x over previous
import functools

import jax
import jax.numpy as jnp
from jax import lax
from jax.experimental import pallas as pl
from jax.experimental.pallas import tpu as pltpu
from jax.experimental.pallas import tpu_sc as plsc

_N = 500
_T = 2
_HD = 256
_EX = 128
_E = 16000
_NP = 512
_P = _NP * _NP
_NSETS = 4
_NC = 2
_NS = 16
_EPW = _E // _NS
_EPW_PAD = 1024
_CHUNK = _P // _NS


def _densify_sc(flat_idx, flat_w, zeros_chunk):
    mesh = plsc.VectorSubcoreMesh(
        core_axis_name="c", subcore_axis_name="s", num_cores=_NC,
        num_subcores=_NS)

    @functools.partial(
        pl.kernel,
        out_type=jax.ShapeDtypeStruct((_NSETS, _P), jnp.float32),
        mesh=mesh,
        scratch_types=[
            pltpu.VMEM((8, 128), jnp.int32),
            pltpu.VMEM((8, 128), jnp.float32),
            pltpu.VMEM_SHARED((_P,), jnp.float32),
            pltpu.VMEM_SHARED((_P,), jnp.float32),
        ],
    )
    def body(idx_hbm, w_hbm, z_hbm, out_hbm, idx_v, w_v, sh0, sh1):
        c = lax.axis_index("c")
        s = lax.axis_index("s")
        for j, sh in enumerate((sh0, sh1)):
            set_id = c * 2 + j
            pltpu.sync_copy(z_hbm, sh.at[pl.ds(s * _CHUNK, _CHUNK)])
            pltpu.sync_copy(idx_hbm.at[set_id, s], idx_v)
            pltpu.sync_copy(w_hbm.at[set_id, s], w_v)
            plsc.subcore_barrier()
            for k in range(8):
                pltpu.sync_copy(w_v.at[k], sh.at[idx_v.at[k]], add=True)
            plsc.subcore_barrier()
            pltpu.sync_copy(
                sh.at[pl.ds(s * _CHUNK, _CHUNK)],
                out_hbm.at[set_id, pl.ds(s * _CHUNK, _CHUNK)])

    return body(flat_idx, flat_w, zeros_chunk)


def _softplus(x):
    return jnp.log1p(jnp.exp(-jnp.abs(x))) + jnp.maximum(x, 0.0)


def _prelu(x, a):
    return jnp.where(x >= 0, x, a * x)


_PREP_W = [
    "phi_x_W", "phi_x_b", "phi_d_W", "phi_d_b", "phi_e_x_W", "phi_e_x_b",
    "enc_W", "enc_b", "Wms", "bms", "sup_enc_W", "sup_enc_b", "Wmss", "bmss",
    "zin1_W", "zin1_b", "zin2_W", "zin2_b", "zout1_W", "zout1_b",
    "zout2_W", "zout2_b", "dec1_Wo", "dec1_Wi", "dec1_b",
    "Wxzr", "Whzr", "bzr", "rnn_xh_W", "rnn_hh_W", "bh",
    "zin_a1", "zin_a2", "zout_a1", "zout_a2",
]


def _prep_body(*refs):
    dense_ref, x_ref, epsx_ref, epsd_ref = refs[:4]
    rest = refs[4:]
    w = dict(zip(_PREP_W, rest[:len(_PREP_W)]))
    a_ref, b_ref = rest[len(_PREP_W):]

    def mm(a, b):
        return jnp.dot(a, b, preferred_element_type=jnp.float32)

    rows = lax.broadcasted_iota(jnp.int32, (_NP, _NP), 0)
    cols = lax.broadcasted_iota(jnp.int32, (_NP, _NP), 1)
    diag = (rows == cols).astype(jnp.float32)

    def norm_adj(S):
        deg = jnp.sum(S, axis=1) + 1.0
        dinv = lax.rsqrt(deg)
        return (S + diag) * dinv[:, None] * dinv[None, :]

    h = jnp.zeros((_NP, _HD), jnp.float32)
    for t in range(_T):
        Ahat = norm_adj(dense_ref[2 * t])
        Ahats = norm_adj(dense_ref[2 * t + 1])
        xt = x_ref[t]
        phi_x = jax.nn.relu(mm(xt, w["phi_x_W"][...]) + w["phi_x_b"][...])
        phi_d = jax.nn.relu(mm(xt, w["phi_d_W"][...]) + w["phi_d_b"][...])
        cat1 = jnp.concatenate([phi_x, h], axis=1)
        enc = jax.nn.relu(mm(Ahat, mm(cat1, w["enc_W"][...])) + w["enc_b"][...])
        ms = mm(Ahat, mm(enc, w["Wms"][...])) + w["bms"][...]
        enc_mean, enc_std = ms[:, :_EX], _softplus(ms[:, _EX:])
        encd = jax.nn.relu(
            mm(Ahats, mm(phi_d, w["sup_enc_W"][...])) + w["sup_enc_b"][...])
        msd = mm(Ahats, mm(encd, w["Wmss"][...])) + w["bmss"][...]
        encd_mean, encd_std = msd[:, :_EX], _softplus(msd[:, _EX:])
        e_x = enc_mean + enc_std * epsx_ref[t]
        e_d = encd_mean + encd_std * epsd_ref[t]
        phi_e_x = jax.nn.relu(
            mm(e_x, w["phi_e_x_W"][...]) + w["phi_e_x_b"][...])
        ed = jnp.concatenate([e_x, e_d], axis=1)
        zi = _prelu(mm(ed, w["zin1_W"][...]) + w["zin1_b"][...],
                    w["zin_a1"][0, 0])
        zi = _prelu(mm(zi, w["zin2_W"][...]) + w["zin2_b"][...],
                    w["zin_a2"][0, 0])
        zo = _prelu(mm(ed, w["zout1_W"][...]) + w["zout1_b"][...],
                    w["zout_a1"][0, 0])
        zo = _prelu(mm(zo, w["zout2_W"][...]) + w["zout2_b"][...],
                    w["zout_a2"][0, 0])
        a_ref[t] = mm(zo, w["dec1_Wo"][...]) + w["dec1_b"][...]
        b_ref[t] = mm(zi, w["dec1_Wi"][...])
        inp = jnp.concatenate([phi_x, phi_e_x], axis=1)
        zr = jax.nn.sigmoid(
            mm(Ahat, mm(inp, w["Wxzr"][...]) + mm(h, w["Whzr"][...]))
            + w["bzr"][...])
        z_g, r_g = zr[:, :_HD], zr[:, _HD:]
        pre_h = mm(Ahat, mm(inp, w["rnn_xh_W"][...])
                   + mm(r_g * h, w["rnn_hh_W"][...])) + w["bh"][...]
        h = z_g * h + (1.0 - z_g) * jnp.tanh(pre_h)


_BI = 8


def _dec_body(a_ref, b_ref, w2_ref, b2_ref, w3_ref, b3_ref, a1_ref, a2_ref,
              out_ref):
    a = a_ref[0]
    b = b_ref[0]
    a1 = a1_ref[0, 0]
    a2 = a2_ref[0, 0]
    h1 = _prelu(a[:, None, :] + b[None, :, :], a1).reshape(_BI * _NP, _HD)
    h2 = _prelu(jnp.dot(h1, w2_ref[...], preferred_element_type=jnp.float32)
                + b2_ref[...], a2)
    o = jax.nn.sigmoid(
        jnp.dot(h2, w3_ref[...], preferred_element_type=jnp.float32)
        + b3_ref[0, 0])
    out_ref[0] = o.reshape(_BI, _NP)


def kernel(x, edge_index, edge_weight, edge_index_sup, edge_weight_sup,
           mask, A_scaler, truths, eps_x, eps_d, h0, params):
    p = params
    f32 = jnp.float32

    def flat(ei, ew):
        src = ei[:, 0, :].astype(jnp.int32)
        dst = ei[:, 1, :].astype(jnp.int32)
        return dst * _NP + src, ew.astype(f32)

    fi_m, fw_m = flat(edge_index, edge_weight)
    fi_s, fw_s = flat(edge_index_sup, edge_weight_sup)
    fi = jnp.stack([fi_m[0], fi_s[0], fi_m[1], fi_s[1]])
    fw = jnp.stack([fw_m[0], fw_s[0], fw_m[1], fw_s[1]])
    pad = _EPW_PAD - _EPW
    fi = jnp.pad(fi.reshape(_NSETS, _NS, _EPW), ((0, 0), (0, 0), (0, pad)))
    fw = jnp.pad(fw.reshape(_NSETS, _NS, _EPW), ((0, 0), (0, 0), (0, pad)))
    fi = fi.reshape(_NSETS, _NS, 8, 128)
    fw = fw.reshape(_NSETS, _NS, 8, 128)
    zeros_chunk = jnp.zeros((_CHUNK,), f32)

    dense = _densify_sc(fi, fw, zeros_chunk).reshape(_NSETS, _NP, _NP)

    padn = _NP - _N
    x_p = jnp.pad(x, ((0, 0), (0, padn), (0, 0)))
    epsx_p = jnp.pad(eps_x, ((0, 0), (0, padn), (0, 0)))
    epsd_p = jnp.pad(eps_d, ((0, 0), (0, padn), (0, 0)))

    def row(v):
        return v.reshape(1, -1)

    wvals = {
        "phi_x_W": p["phi_x_W"], "phi_x_b": row(p["phi_x_b"]),
        "phi_d_W": p["phi_d_W"], "phi_d_b": row(p["phi_d_b"]),
        "phi_e_x_W": p["phi_e_x_W"], "phi_e_x_b": row(p["phi_e_x_b"]),
        "enc_W": p["enc_W"], "enc_b": row(p["enc_b"]),
        "Wms": jnp.concatenate([p["enc_mean_W"], p["enc_std_W"]], 1),
        "bms": row(jnp.concatenate([p["enc_mean_b"], p["enc_std_b"]])),
        "sup_enc_W": p["sup_enc_W"], "sup_enc_b": row(p["sup_enc_b"]),
        "Wmss": jnp.concatenate([p["sup_enc_mean_W"], p["sup_enc_std_W"]], 1),
        "bmss": row(jnp.concatenate([p["sup_enc_mean_b"],
                                     p["sup_enc_std_b"]])),
        "zin1_W": p["zin1_W"], "zin1_b": row(p["zin1_b"]),
        "zin2_W": p["zin2_W"], "zin2_b": row(p["zin2_b"]),
        "zout1_W": p["zout1_W"], "zout1_b": row(p["zout1_b"]),
        "zout2_W": p["zout2_W"], "zout2_b": row(p["zout2_b"]),
        "dec1_Wo": p["dec1_W"][:64], "dec1_Wi": p["dec1_W"][64:],
        "dec1_b": row(p["dec1_b"]),
        "Wxzr": jnp.concatenate([p["rnn_xz_W"], p["rnn_xr_W"]], 1),
        "Whzr": jnp.concatenate([p["rnn_hz_W"], p["rnn_hr_W"]], 1),
        "bzr": row(jnp.concatenate([p["rnn_xz_b"] + p["rnn_hz_b"],
                                    p["rnn_xr_b"] + p["rnn_hr_b"]])),
        "rnn_xh_W": p["rnn_xh_W"], "rnn_hh_W": p["rnn_hh_W"],
        "bh": row(p["rnn_xh_b"] + p["rnn_hh_b"]),
        "zin_a1": p["zin_a1"].reshape(1, 1),
        "zin_a2": p["zin_a2"].reshape(1, 1),
        "zout_a1": p["zout_a1"].reshape(1, 1),
        "zout_a2": p["zout_a2"].reshape(1, 1),
    }
    prep_in = [dense, x_p, epsx_p, epsd_p] + [wvals[n] for n in _PREP_W]

    A, B = pl.pallas_call(
        _prep_body,
        out_shape=[
            jax.ShapeDtypeStruct((_T, _NP, _HD), f32),
            jax.ShapeDtypeStruct((_T, _NP, _HD), f32),
        ],
    )(*prep_in)

    out = pl.pallas_call(
        _dec_body,
        grid=(_T, _NP // _BI),
        in_specs=[
            pl.BlockSpec((1, _BI, _HD), lambda t, i: (t, i, 0)),
            pl.BlockSpec((1, _NP, _HD), lambda t, i: (t, 0, 0)),
            pl.BlockSpec((_HD, _HD), lambda t, i: (0, 0)),
            pl.BlockSpec((1, _HD), lambda t, i: (0, 0)),
            pl.BlockSpec((_HD, 1), lambda t, i: (0, 0)),
            pl.BlockSpec((1, 1), lambda t, i: (0, 0)),
            pl.BlockSpec((1, 1), lambda t, i: (0, 0)),
            pl.BlockSpec((1, 1), lambda t, i: (0, 0)),
        ],
        out_specs=pl.BlockSpec((1, _BI, _NP), lambda t, i: (t, i, 0)),
        out_shape=jax.ShapeDtypeStruct((_T, _NP, _NP), f32),
        compiler_params=pltpu.CompilerParams(
            dimension_semantics=("parallel", "parallel")),
    )(A, B, p["dec2_W"], row(p["dec2_b"]), p["dec3_W"],
      p["dec3_b"].reshape(1, 1), p["dec_a1"].reshape(1, 1),
      p["dec_a2"].reshape(1, 1))

    return out[:, :_N, :_N]


# speedup vs baseline: 14.4481x; 1.1798x over previous
import functools

import jax
import jax.numpy as jnp
from jax import lax
from jax.experimental import pallas as pl
from jax.experimental.pallas import tpu as pltpu
from jax.experimental.pallas import tpu_sc as plsc

_N = 500
_T = 2
_HD = 256
_EX = 128
_E = 16000
_NP = 512
_P = _NP * _NP
_NSETS = 4
_NC = 2
_NS = 16
_EPW = _E // _NS
_EPW_PAD = 1024
_CHUNK = _P // _NS


def _densify_sc(flat_idx, flat_w, zeros_chunk):
    mesh = plsc.VectorSubcoreMesh(
        core_axis_name="c", subcore_axis_name="s", num_cores=_NC,
        num_subcores=_NS)

    @functools.partial(
        pl.kernel,
        out_type=jax.ShapeDtypeStruct((_NSETS, _P), jnp.float32),
        mesh=mesh,
        scratch_types=[
            pltpu.VMEM((8, 128), jnp.int32),
            pltpu.VMEM((8, 128), jnp.float32),
            pltpu.VMEM_SHARED((_P,), jnp.float32),
            pltpu.VMEM_SHARED((_P,), jnp.float32),
        ],
    )
    def body(idx_hbm, w_hbm, z_hbm, out_hbm, idx_v, w_v, sh0, sh1):
        c = lax.axis_index("c")
        s = lax.axis_index("s")
        for j, sh in enumerate((sh0, sh1)):
            set_id = c * 2 + j
            pltpu.sync_copy(z_hbm, sh.at[pl.ds(s * _CHUNK, _CHUNK)])
            pltpu.sync_copy(idx_hbm.at[set_id, s], idx_v)
            pltpu.sync_copy(w_hbm.at[set_id, s], w_v)
            plsc.subcore_barrier()
            for k in range(8):
                pltpu.sync_copy(w_v.at[k], sh.at[idx_v.at[k]], add=True)
            plsc.subcore_barrier()
            pltpu.sync_copy(
                sh.at[pl.ds(s * _CHUNK, _CHUNK)],
                out_hbm.at[set_id, pl.ds(s * _CHUNK, _CHUNK)])

    return body(flat_idx, flat_w, zeros_chunk)


def _softplus(x):
    return jnp.log1p(jnp.exp(-jnp.abs(x))) + jnp.maximum(x, 0.0)


def _prelu(x, a):
    return jnp.where(x >= 0, x, a * x)


_PREP_W = [
    "phi_x_W", "phi_x_b", "phi_d_W", "phi_d_b", "phi_e_x_W", "phi_e_x_b",
    "enc_W", "enc_b", "Wms", "bms", "sup_enc_W", "sup_enc_b", "Wmss", "bmss",
    "zin1_W", "zin1_b", "zin2_W", "zin2_b", "zout1_W", "zout1_b",
    "zout2_W", "zout2_b", "dec1_Wo", "dec1_Wi", "dec1_b",
    "Wxzr", "Whzr", "bzr", "rnn_xh_W", "rnn_hh_W", "bh",
    "zin_a1", "zin_a2", "zout_a1", "zout_a2",
    "dec2_W", "dec2_b", "dec_a1", "dec3_W", "dec3_Wr",
]


def _prep_body(*refs):
    dense_ref, x_ref, epsx_ref, epsd_ref = refs[:4]
    rest = refs[4:]
    w = dict(zip(_PREP_W, rest[:len(_PREP_W)]))
    a_ref, b_ref, c_ref, d_ref, cv_ref, dv_ref = rest[len(_PREP_W):]

    def mm(a, b):
        return jnp.dot(a, b, preferred_element_type=jnp.float32)

    rows = lax.broadcasted_iota(jnp.int32, (_NP, _NP), 0)
    cols = lax.broadcasted_iota(jnp.int32, (_NP, _NP), 1)
    diag = (rows == cols).astype(jnp.float32)

    def norm_adj(S):
        deg = jnp.sum(S, axis=1) + 1.0
        dinv = lax.rsqrt(deg)
        return (S + diag) * dinv[:, None] * dinv[None, :]

    h = jnp.zeros((_NP, _HD), jnp.float32)
    for t in range(_T):
        Ahat = norm_adj(dense_ref[2 * t])
        Ahats = norm_adj(dense_ref[2 * t + 1])
        xt = x_ref[t]
        phi_x = jax.nn.relu(mm(xt, w["phi_x_W"][...]) + w["phi_x_b"][...])
        phi_d = jax.nn.relu(mm(xt, w["phi_d_W"][...]) + w["phi_d_b"][...])
        cat1 = jnp.concatenate([phi_x, h], axis=1)
        enc = jax.nn.relu(mm(Ahat, mm(cat1, w["enc_W"][...])) + w["enc_b"][...])
        ms = mm(Ahat, mm(enc, w["Wms"][...])) + w["bms"][...]
        enc_mean, enc_std = ms[:, :_EX], _softplus(ms[:, _EX:])
        encd = jax.nn.relu(
            mm(Ahats, mm(phi_d, w["sup_enc_W"][...])) + w["sup_enc_b"][...])
        msd = mm(Ahats, mm(encd, w["Wmss"][...])) + w["bmss"][...]
        encd_mean, encd_std = msd[:, :_EX], _softplus(msd[:, _EX:])
        e_x = enc_mean + enc_std * epsx_ref[t]
        e_d = encd_mean + encd_std * epsd_ref[t]
        phi_e_x = jax.nn.relu(
            mm(e_x, w["phi_e_x_W"][...]) + w["phi_e_x_b"][...])
        ed = jnp.concatenate([e_x, e_d], axis=1)
        zi = _prelu(mm(ed, w["zin1_W"][...]) + w["zin1_b"][...],
                    w["zin_a1"][0, 0])
        zi = _prelu(mm(zi, w["zin2_W"][...]) + w["zin2_b"][...],
                    w["zin_a2"][0, 0])
        zo = _prelu(mm(ed, w["zout1_W"][...]) + w["zout1_b"][...],
                    w["zout_a1"][0, 0])
        zo = _prelu(mm(zo, w["zout2_W"][...]) + w["zout2_b"][...],
                    w["zout_a2"][0, 0])
        av = mm(zo, w["dec1_Wo"][...]) + w["dec1_b"][...]
        bv = mm(zi, w["dec1_Wi"][...])
        a_ref[t] = av
        b_ref[t] = bv
        a1 = w["dec_a1"][0, 0]
        cval = a1 * mm(av, w["dec2_W"][...]) + w["dec2_b"][...]
        dval = a1 * mm(bv, w["dec2_W"][...])
        c_ref[t] = cval
        d_ref[t] = dval
        cv_ref[t] = mm(cval, w["dec3_W"][...])
        dv_ref[t] = lax.dot_general(
            w["dec3_Wr"][...], dval, (((1,), (1,)), ((), ())),
            preferred_element_type=jnp.float32)
        inp = jnp.concatenate([phi_x, phi_e_x], axis=1)
        zr = jax.nn.sigmoid(
            mm(Ahat, mm(inp, w["Wxzr"][...]) + mm(h, w["Whzr"][...]))
            + w["bzr"][...])
        z_g, r_g = zr[:, :_HD], zr[:, _HD:]
        pre_h = mm(Ahat, mm(inp, w["rnn_xh_W"][...])
                   + mm(r_g * h, w["rnn_hh_W"][...])) + w["bh"][...]
        h = z_g * h + (1.0 - z_g) * jnp.tanh(pre_h)


_BI = 8


def _dec_body(a_ref, b_ref, c_ref, d_ref, cv_ref, dv_ref, w2p_ref, u_ref,
              w3r_ref, b3_ref, a2_ref, out_ref):
    a = a_ref[0]
    b = b_ref[0]
    a2 = a2_ref[0, 0]
    r = jnp.maximum(a[:, None, :] + b[None, :, :], 0.0).reshape(
        _BI * _NP, _HD)
    m = jnp.dot(r, w2p_ref[...], preferred_element_type=jnp.float32)
    q = m.reshape(_BI, _NP, _HD) + c_ref[0][:, None, :] + d_ref[0][None, :, :]
    h2 = _prelu(q, a2).reshape(_BI * _NP, _HD)
    s = lax.dot_general(w3r_ref[...], h2, (((1,), (1,)), ((), ())),
                        preferred_element_type=jnp.float32)
    pre = s.reshape(_BI, _NP) + b3_ref[0, 0]
    out_ref[0] = jax.nn.sigmoid(pre)


def kernel(x, edge_index, edge_weight, edge_index_sup, edge_weight_sup,
           mask, A_scaler, truths, eps_x, eps_d, h0, params):
    p = params
    f32 = jnp.float32

    def flat(ei, ew):
        src = ei[:, 0, :].astype(jnp.int32)
        dst = ei[:, 1, :].astype(jnp.int32)
        return dst * _NP + src, ew.astype(f32)

    fi_m, fw_m = flat(edge_index, edge_weight)
    fi_s, fw_s = flat(edge_index_sup, edge_weight_sup)
    fi = jnp.stack([fi_m[0], fi_s[0], fi_m[1], fi_s[1]])
    fw = jnp.stack([fw_m[0], fw_s[0], fw_m[1], fw_s[1]])
    pad = _EPW_PAD - _EPW
    fi = jnp.pad(fi.reshape(_NSETS, _NS, _EPW), ((0, 0), (0, 0), (0, pad)))
    fw = jnp.pad(fw.reshape(_NSETS, _NS, _EPW), ((0, 0), (0, 0), (0, pad)))
    fi = fi.reshape(_NSETS, _NS, 8, 128)
    fw = fw.reshape(_NSETS, _NS, 8, 128)
    zeros_chunk = jnp.zeros((_CHUNK,), f32)

    dense = _densify_sc(fi, fw, zeros_chunk).reshape(_NSETS, _NP, _NP)

    padn = _NP - _N
    x_p = jnp.pad(x, ((0, 0), (0, padn), (0, 0)))
    epsx_p = jnp.pad(eps_x, ((0, 0), (0, padn), (0, 0)))
    epsd_p = jnp.pad(eps_d, ((0, 0), (0, padn), (0, 0)))

    def row(v):
        return v.reshape(1, -1)

    wvals = {
        "phi_x_W": p["phi_x_W"], "phi_x_b": row(p["phi_x_b"]),
        "phi_d_W": p["phi_d_W"], "phi_d_b": row(p["phi_d_b"]),
        "phi_e_x_W": p["phi_e_x_W"], "phi_e_x_b": row(p["phi_e_x_b"]),
        "enc_W": p["enc_W"], "enc_b": row(p["enc_b"]),
        "Wms": jnp.concatenate([p["enc_mean_W"], p["enc_std_W"]], 1),
        "bms": row(jnp.concatenate([p["enc_mean_b"], p["enc_std_b"]])),
        "sup_enc_W": p["sup_enc_W"], "sup_enc_b": row(p["sup_enc_b"]),
        "Wmss": jnp.concatenate([p["sup_enc_mean_W"], p["sup_enc_std_W"]], 1),
        "bmss": row(jnp.concatenate([p["sup_enc_mean_b"],
                                     p["sup_enc_std_b"]])),
        "zin1_W": p["zin1_W"], "zin1_b": row(p["zin1_b"]),
        "zin2_W": p["zin2_W"], "zin2_b": row(p["zin2_b"]),
        "zout1_W": p["zout1_W"], "zout1_b": row(p["zout1_b"]),
        "zout2_W": p["zout2_W"], "zout2_b": row(p["zout2_b"]),
        "dec1_Wo": p["dec1_W"][:64], "dec1_Wi": p["dec1_W"][64:],
        "dec1_b": row(p["dec1_b"]),
        "Wxzr": jnp.concatenate([p["rnn_xz_W"], p["rnn_xr_W"]], 1),
        "Whzr": jnp.concatenate([p["rnn_hz_W"], p["rnn_hr_W"]], 1),
        "bzr": row(jnp.concatenate([p["rnn_xz_b"] + p["rnn_hz_b"],
                                    p["rnn_xr_b"] + p["rnn_hr_b"]])),
        "rnn_xh_W": p["rnn_xh_W"], "rnn_hh_W": p["rnn_hh_W"],
        "bh": row(p["rnn_xh_b"] + p["rnn_hh_b"]),
        "zin_a1": p["zin_a1"].reshape(1, 1),
        "zin_a2": p["zin_a2"].reshape(1, 1),
        "zout_a1": p["zout_a1"].reshape(1, 1),
        "zout_a2": p["zout_a2"].reshape(1, 1),
        "dec2_W": p["dec2_W"], "dec2_b": row(p["dec2_b"]),
        "dec_a1": p["dec_a1"].reshape(1, 1),
        "dec3_W": p["dec3_W"], "dec3_Wr": p["dec3_W"].reshape(1, _HD),
    }
    prep_in = [dense, x_p, epsx_p, epsd_p] + [wvals[n] for n in _PREP_W]

    A, B, C, D, CV, DV = pl.pallas_call(
        _prep_body,
        out_shape=[
            jax.ShapeDtypeStruct((_T, _NP, _HD), f32),
            jax.ShapeDtypeStruct((_T, _NP, _HD), f32),
            jax.ShapeDtypeStruct((_T, _NP, _HD), f32),
            jax.ShapeDtypeStruct((_T, _NP, _HD), f32),
            jax.ShapeDtypeStruct((_T, _NP, 1), f32),
            jax.ShapeDtypeStruct((_T, 1, _NP), f32),
        ],
    )(*prep_in)

    w2p = (1.0 - p["dec_a1"]) * p["dec2_W"]
    u_row = jnp.dot(w2p, p["dec3_W"]).reshape(1, _HD)
    out = pl.pallas_call(
        _dec_body,
        grid=(_T, _NP // _BI),
        in_specs=[
            pl.BlockSpec((1, _BI, _HD), lambda t, i: (t, i, 0)),
            pl.BlockSpec((1, _NP, _HD), lambda t, i: (t, 0, 0)),
            pl.BlockSpec((1, _BI, _HD), lambda t, i: (t, i, 0)),
            pl.BlockSpec((1, _NP, _HD), lambda t, i: (t, 0, 0)),
            pl.BlockSpec((1, _BI, 1), lambda t, i: (t, i, 0)),
            pl.BlockSpec((1, 1, _NP), lambda t, i: (t, 0, 0)),
            pl.BlockSpec((_HD, _HD), lambda t, i: (0, 0)),
            pl.BlockSpec((1, _HD), lambda t, i: (0, 0)),
            pl.BlockSpec((1, _HD), lambda t, i: (0, 0)),
            pl.BlockSpec((1, 1), lambda t, i: (0, 0)),
            pl.BlockSpec((1, 1), lambda t, i: (0, 0)),
        ],
        out_specs=pl.BlockSpec((1, _BI, _NP), lambda t, i: (t, i, 0)),
        out_shape=jax.ShapeDtypeStruct((_T, _NP, _NP), f32),
        compiler_params=pltpu.CompilerParams(
            dimension_semantics=("parallel", "parallel")),
    )(A, B, C, D, CV, DV, w2p, u_row, p["dec3_W"].reshape(1, _HD),
      p["dec3_b"].reshape(1, 1), p["dec_a2"].reshape(1, 1))

    return out[:, :_N, :_N]


# speedup vs baseline: 16.6816x; 1.1546x over previous
import functools

import jax
import jax.numpy as jnp
from jax import lax
from jax.experimental import pallas as pl
from jax.experimental.pallas import tpu as pltpu
from jax.experimental.pallas import tpu_sc as plsc

_N = 500
_T = 2
_HD = 256
_EX = 128
_E = 16000
_NP = 512
_P = _NP * _NP
_NSETS = 4
_NC = 2
_NS = 16
_EPW = _E // _NS
_EPW_PAD = 1024
_CHUNK = _P // _NS


def _densify_sc(flat_idx, flat_w, zeros_chunk):
    mesh = plsc.VectorSubcoreMesh(
        core_axis_name="c", subcore_axis_name="s", num_cores=_NC,
        num_subcores=_NS)

    @functools.partial(
        pl.kernel,
        out_type=jax.ShapeDtypeStruct((_NSETS, _P), jnp.float32),
        mesh=mesh,
        scratch_types=[
            pltpu.VMEM((8, 128), jnp.int32),
            pltpu.VMEM((8, 128), jnp.float32),
            pltpu.VMEM_SHARED((_P,), jnp.float32),
            pltpu.VMEM_SHARED((_P,), jnp.float32),
        ],
    )
    def body(idx_hbm, w_hbm, z_hbm, out_hbm, idx_v, w_v, sh0, sh1):
        c = lax.axis_index("c")
        s = lax.axis_index("s")
        for j, sh in enumerate((sh0, sh1)):
            set_id = c * 2 + j
            pltpu.sync_copy(z_hbm, sh.at[pl.ds(s * _CHUNK, _CHUNK)])
            pltpu.sync_copy(idx_hbm.at[set_id, s], idx_v)
            pltpu.sync_copy(w_hbm.at[set_id, s], w_v)
            plsc.subcore_barrier()
            for k in range(8):
                pltpu.sync_copy(w_v.at[k], sh.at[idx_v.at[k]], add=True)
            plsc.subcore_barrier()
            pltpu.sync_copy(
                sh.at[pl.ds(s * _CHUNK, _CHUNK)],
                out_hbm.at[set_id, pl.ds(s * _CHUNK, _CHUNK)])

    return body(flat_idx, flat_w, zeros_chunk)


def _softplus(x):
    return jnp.log1p(jnp.exp(-jnp.abs(x))) + jnp.maximum(x, 0.0)


def _prelu(x, a):
    return jnp.where(x >= 0, x, a * x)


_PREP_W = [
    "phi_x_W", "phi_x_b", "phi_d_W", "phi_d_b", "phi_e_x_W", "phi_e_x_b",
    "enc_W", "enc_b", "Wms", "bms", "sup_enc_W", "sup_enc_b", "Wmss", "bmss",
    "zin1_W", "zin1_b", "zin2_W", "zin2_b", "zout1_W", "zout1_b",
    "zout2_W", "zout2_b", "dec1_Wo", "dec1_Wi", "dec1_b",
    "Wxzr", "Whzr", "bzr", "rnn_xh_W", "rnn_hh_W", "bh",
    "zin_a1", "zin_a2", "zout_a1", "zout_a2",
    "dec2_W", "dec2_b", "dec_a1", "dec3_W", "dec3_Wr",
]


def _prep_body(*refs):
    dense_ref, x_ref, epsx_ref, epsd_ref = refs[:4]
    rest = refs[4:]
    w = dict(zip(_PREP_W, rest[:len(_PREP_W)]))
    a_ref, b_ref, c_ref, d_ref, cv_ref, dv_ref = rest[len(_PREP_W):]

    def mm(a, b):
        return jnp.dot(a, b, preferred_element_type=jnp.float32)

    rows = lax.broadcasted_iota(jnp.int32, (_NP, _NP), 0)
    cols = lax.broadcasted_iota(jnp.int32, (_NP, _NP), 1)
    diag = (rows == cols).astype(jnp.float32)

    def norm_adj(S):
        deg = jnp.sum(S, axis=1) + 1.0
        dinv = lax.rsqrt(deg)
        return (S + diag) * dinv[:, None] * dinv[None, :]

    h = jnp.zeros((_NP, _HD), jnp.float32)
    for t in range(_T):
        Ahat = norm_adj(dense_ref[2 * t])
        Ahats = norm_adj(dense_ref[2 * t + 1])
        xt = x_ref[t]
        phi_x = jax.nn.relu(mm(xt, w["phi_x_W"][...]) + w["phi_x_b"][...])
        phi_d = jax.nn.relu(mm(xt, w["phi_d_W"][...]) + w["phi_d_b"][...])
        cat1 = jnp.concatenate([phi_x, h], axis=1)
        enc = jax.nn.relu(mm(Ahat, mm(cat1, w["enc_W"][...])) + w["enc_b"][...])
        ms = mm(Ahat, mm(enc, w["Wms"][...])) + w["bms"][...]
        enc_mean, enc_std = ms[:, :_EX], _softplus(ms[:, _EX:])
        encd = jax.nn.relu(
            mm(Ahats, mm(phi_d, w["sup_enc_W"][...])) + w["sup_enc_b"][...])
        msd = mm(Ahats, mm(encd, w["Wmss"][...])) + w["bmss"][...]
        encd_mean, encd_std = msd[:, :_EX], _softplus(msd[:, _EX:])
        e_x = enc_mean + enc_std * epsx_ref[t]
        e_d = encd_mean + encd_std * epsd_ref[t]
        phi_e_x = jax.nn.relu(
            mm(e_x, w["phi_e_x_W"][...]) + w["phi_e_x_b"][...])
        ed = jnp.concatenate([e_x, e_d], axis=1)
        zi = _prelu(mm(ed, w["zin1_W"][...]) + w["zin1_b"][...],
                    w["zin_a1"][0, 0])
        zi = _prelu(mm(zi, w["zin2_W"][...]) + w["zin2_b"][...],
                    w["zin_a2"][0, 0])
        zo = _prelu(mm(ed, w["zout1_W"][...]) + w["zout1_b"][...],
                    w["zout_a1"][0, 0])
        zo = _prelu(mm(zo, w["zout2_W"][...]) + w["zout2_b"][...],
                    w["zout_a2"][0, 0])
        av = mm(zo, w["dec1_Wo"][...]) + w["dec1_b"][...]
        bv = mm(zi, w["dec1_Wi"][...])
        a_ref[t] = av
        b_ref[t] = bv
        a1 = w["dec_a1"][0, 0]
        cval = a1 * mm(av, w["dec2_W"][...]) + w["dec2_b"][...]
        dval = a1 * mm(bv, w["dec2_W"][...])
        c_ref[t] = cval
        d_ref[t] = dval
        cv_ref[t] = mm(cval, w["dec3_W"][...])
        dv_ref[t] = lax.dot_general(
            w["dec3_Wr"][...], dval, (((1,), (1,)), ((), ())),
            preferred_element_type=jnp.float32)
        inp = jnp.concatenate([phi_x, phi_e_x], axis=1)
        zr = jax.nn.sigmoid(
            mm(Ahat, mm(inp, w["Wxzr"][...]) + mm(h, w["Whzr"][...]))
            + w["bzr"][...])
        z_g, r_g = zr[:, :_HD], zr[:, _HD:]
        pre_h = mm(Ahat, mm(inp, w["rnn_xh_W"][...])
                   + mm(r_g * h, w["rnn_hh_W"][...])) + w["bh"][...]
        h = z_g * h + (1.0 - z_g) * jnp.tanh(pre_h)


_BI = 32


def _dec_body(a_ref, b_ref, c_ref, d_ref, cv_ref, dv_ref, w2p_ref, u_ref,
              w3r_ref, b3_ref, a2_ref, out_ref):
    a = a_ref[0]
    b = b_ref[0]
    a2 = a2_ref[0, 0]
    r = jnp.maximum(a[:, None, :] + b[None, :, :], 0.0).reshape(
        _BI * _NP, _HD)
    m = jnp.dot(r, w2p_ref[...], preferred_element_type=jnp.float32)
    q = m.reshape(_BI, _NP, _HD) + c_ref[0][:, None, :] + d_ref[0][None, :, :]
    h2 = _prelu(q, a2).reshape(_BI * _NP, _HD)
    s = lax.dot_general(w3r_ref[...], h2, (((1,), (1,)), ((), ())),
                        preferred_element_type=jnp.float32)
    pre = s.reshape(_BI, _NP) + b3_ref[0, 0]
    out_ref[0] = jax.nn.sigmoid(pre)


def kernel(x, edge_index, edge_weight, edge_index_sup, edge_weight_sup,
           mask, A_scaler, truths, eps_x, eps_d, h0, params):
    p = params
    f32 = jnp.float32

    def flat(ei, ew):
        src = ei[:, 0, :].astype(jnp.int32)
        dst = ei[:, 1, :].astype(jnp.int32)
        return dst * _NP + src, ew.astype(f32)

    fi_m, fw_m = flat(edge_index, edge_weight)
    fi_s, fw_s = flat(edge_index_sup, edge_weight_sup)
    fi = jnp.stack([fi_m[0], fi_s[0], fi_m[1], fi_s[1]])
    fw = jnp.stack([fw_m[0], fw_s[0], fw_m[1], fw_s[1]])
    pad = _EPW_PAD - _EPW
    fi = jnp.pad(fi.reshape(_NSETS, _NS, _EPW), ((0, 0), (0, 0), (0, pad)))
    fw = jnp.pad(fw.reshape(_NSETS, _NS, _EPW), ((0, 0), (0, 0), (0, pad)))
    fi = fi.reshape(_NSETS, _NS, 8, 128)
    fw = fw.reshape(_NSETS, _NS, 8, 128)
    zeros_chunk = jnp.zeros((_CHUNK,), f32)

    dense = _densify_sc(fi, fw, zeros_chunk).reshape(_NSETS, _NP, _NP)

    padn = _NP - _N
    x_p = jnp.pad(x, ((0, 0), (0, padn), (0, 0)))
    epsx_p = jnp.pad(eps_x, ((0, 0), (0, padn), (0, 0)))
    epsd_p = jnp.pad(eps_d, ((0, 0), (0, padn), (0, 0)))

    def row(v):
        return v.reshape(1, -1)

    wvals = {
        "phi_x_W": p["phi_x_W"], "phi_x_b": row(p["phi_x_b"]),
        "phi_d_W": p["phi_d_W"], "phi_d_b": row(p["phi_d_b"]),
        "phi_e_x_W": p["phi_e_x_W"], "phi_e_x_b": row(p["phi_e_x_b"]),
        "enc_W": p["enc_W"], "enc_b": row(p["enc_b"]),
        "Wms": jnp.concatenate([p["enc_mean_W"], p["enc_std_W"]], 1),
        "bms": row(jnp.concatenate([p["enc_mean_b"], p["enc_std_b"]])),
        "sup_enc_W": p["sup_enc_W"], "sup_enc_b": row(p["sup_enc_b"]),
        "Wmss": jnp.concatenate([p["sup_enc_mean_W"], p["sup_enc_std_W"]], 1),
        "bmss": row(jnp.concatenate([p["sup_enc_mean_b"],
                                     p["sup_enc_std_b"]])),
        "zin1_W": p["zin1_W"], "zin1_b": row(p["zin1_b"]),
        "zin2_W": p["zin2_W"], "zin2_b": row(p["zin2_b"]),
        "zout1_W": p["zout1_W"], "zout1_b": row(p["zout1_b"]),
        "zout2_W": p["zout2_W"], "zout2_b": row(p["zout2_b"]),
        "dec1_Wo": p["dec1_W"][:64], "dec1_Wi": p["dec1_W"][64:],
        "dec1_b": row(p["dec1_b"]),
        "Wxzr": jnp.concatenate([p["rnn_xz_W"], p["rnn_xr_W"]], 1),
        "Whzr": jnp.concatenate([p["rnn_hz_W"], p["rnn_hr_W"]], 1),
        "bzr": row(jnp.concatenate([p["rnn_xz_b"] + p["rnn_hz_b"],
                                    p["rnn_xr_b"] + p["rnn_hr_b"]])),
        "rnn_xh_W": p["rnn_xh_W"], "rnn_hh_W": p["rnn_hh_W"],
        "bh": row(p["rnn_xh_b"] + p["rnn_hh_b"]),
        "zin_a1": p["zin_a1"].reshape(1, 1),
        "zin_a2": p["zin_a2"].reshape(1, 1),
        "zout_a1": p["zout_a1"].reshape(1, 1),
        "zout_a2": p["zout_a2"].reshape(1, 1),
        "dec2_W": p["dec2_W"], "dec2_b": row(p["dec2_b"]),
        "dec_a1": p["dec_a1"].reshape(1, 1),
        "dec3_W": p["dec3_W"], "dec3_Wr": p["dec3_W"].reshape(1, _HD),
    }
    prep_in = [dense, x_p, epsx_p, epsd_p] + [wvals[n] for n in _PREP_W]

    A, B, C, D, CV, DV = pl.pallas_call(
        _prep_body,
        out_shape=[
            jax.ShapeDtypeStruct((_T, _NP, _HD), f32),
            jax.ShapeDtypeStruct((_T, _NP, _HD), f32),
            jax.ShapeDtypeStruct((_T, _NP, _HD), f32),
            jax.ShapeDtypeStruct((_T, _NP, _HD), f32),
            jax.ShapeDtypeStruct((_T, _NP, 1), f32),
            jax.ShapeDtypeStruct((_T, 1, _NP), f32),
        ],
    )(*prep_in)

    w2p = (1.0 - p["dec_a1"]) * p["dec2_W"]
    u_row = jnp.dot(w2p, p["dec3_W"]).reshape(1, _HD)
    out = pl.pallas_call(
        _dec_body,
        grid=(_T, _NP // _BI),
        in_specs=[
            pl.BlockSpec((1, _BI, _HD), lambda t, i: (t, i, 0)),
            pl.BlockSpec((1, _NP, _HD), lambda t, i: (t, 0, 0)),
            pl.BlockSpec((1, _BI, _HD), lambda t, i: (t, i, 0)),
            pl.BlockSpec((1, _NP, _HD), lambda t, i: (t, 0, 0)),
            pl.BlockSpec((1, _BI, 1), lambda t, i: (t, i, 0)),
            pl.BlockSpec((1, 1, _NP), lambda t, i: (t, 0, 0)),
            pl.BlockSpec((_HD, _HD), lambda t, i: (0, 0)),
            pl.BlockSpec((1, _HD), lambda t, i: (0, 0)),
            pl.BlockSpec((1, _HD), lambda t, i: (0, 0)),
            pl.BlockSpec((1, 1), lambda t, i: (0, 0)),
            pl.BlockSpec((1, 1), lambda t, i: (0, 0)),
        ],
        out_specs=pl.BlockSpec((1, _BI, _NP), lambda t, i: (t, i, 0)),
        out_shape=jax.ShapeDtypeStruct((_T, _NP, _NP), f32),
        compiler_params=pltpu.CompilerParams(
            dimension_semantics=("parallel", "parallel")),
    )(A, B, C, D, CV, DV, w2p, u_row, p["dec3_W"].reshape(1, _HD),
      p["dec3_b"].reshape(1, 1), p["dec_a2"].reshape(1, 1))

    return out[:, :_N, :_N]


# speedup vs baseline: 16.7779x; 1.0058x over previous
import functools

import jax
import jax.numpy as jnp
from jax import lax
from jax.experimental import pallas as pl
from jax.experimental.pallas import tpu as pltpu
from jax.experimental.pallas import tpu_sc as plsc

_N = 500
_T = 2
_HD = 256
_EX = 128
_E = 16000
_NP = 512
_P = _NP * _NP
_NSETS = 4
_NC = 2
_NS = 16
_EPW = _E // _NS
_EPW_PAD = 1024
_CHUNK = _P // _NS


def _densify_sc(flat_idx, flat_w, zeros_chunk):
    mesh = plsc.VectorSubcoreMesh(
        core_axis_name="c", subcore_axis_name="s", num_cores=_NC,
        num_subcores=_NS)

    @functools.partial(
        pl.kernel,
        out_type=jax.ShapeDtypeStruct((_NSETS, _P), jnp.float32),
        mesh=mesh,
        scratch_types=[
            pltpu.VMEM((8, 128), jnp.int32),
            pltpu.VMEM((8, 128), jnp.float32),
            pltpu.VMEM_SHARED((_P,), jnp.float32),
            pltpu.VMEM_SHARED((_P,), jnp.float32),
        ],
    )
    def body(idx_hbm, w_hbm, z_hbm, out_hbm, idx_v, w_v, sh0, sh1):
        c = lax.axis_index("c")
        s = lax.axis_index("s")
        for j, sh in enumerate((sh0, sh1)):
            set_id = c * 2 + j
            pltpu.sync_copy(z_hbm, sh.at[pl.ds(s * _CHUNK, _CHUNK)])
            pltpu.sync_copy(idx_hbm.at[set_id, s], idx_v)
            pltpu.sync_copy(w_hbm.at[set_id, s], w_v)
            plsc.subcore_barrier()
            for k in range(8):
                pltpu.sync_copy(w_v.at[k], sh.at[idx_v.at[k]], add=True)
            plsc.subcore_barrier()
            pltpu.sync_copy(
                sh.at[pl.ds(s * _CHUNK, _CHUNK)],
                out_hbm.at[set_id, pl.ds(s * _CHUNK, _CHUNK)])

    return body(flat_idx, flat_w, zeros_chunk)


def _softplus(x):
    return jnp.log1p(jnp.exp(-jnp.abs(x))) + jnp.maximum(x, 0.0)


def _prelu(x, a):
    return jnp.where(x >= 0, x, a * x)


_PREP_W = [
    "phi_x_W", "phi_x_b", "phi_d_W", "phi_d_b", "phi_e_x_W", "phi_e_x_b",
    "enc_W", "enc_b", "Wms", "bms", "sup_enc_W", "sup_enc_b", "Wmss", "bmss",
    "zin1_W", "zin1_b", "zin2_W", "zin2_b", "zout1_W", "zout1_b",
    "zout2_W", "zout2_b", "dec1_Wo", "dec1_Wi", "dec1_b",
    "Wxzr", "Whzr", "bzr", "rnn_xh_W", "rnn_hh_W", "bh",
    "zin_a1", "zin_a2", "zout_a1", "zout_a2",
    "dec2_W", "dec2_b", "dec_a1", "dec3_W", "dec3_Wr",
]


def _prep_body(*refs):
    dense_ref, x_ref, epsx_ref, epsd_ref = refs[:4]
    rest = refs[4:]
    w = dict(zip(_PREP_W, rest[:len(_PREP_W)]))
    a_ref, b_ref, c_ref, d_ref = rest[len(_PREP_W):]

    def mm(a, b):
        return jnp.dot(a, b, preferred_element_type=jnp.float32)

    rows = lax.broadcasted_iota(jnp.int32, (_NP, _NP), 0)
    cols = lax.broadcasted_iota(jnp.int32, (_NP, _NP), 1)
    diag = (rows == cols).astype(jnp.float32)

    def norm_adj(S):
        deg = jnp.sum(S, axis=1) + 1.0
        dinv = lax.rsqrt(deg)
        return (S + diag) * dinv[:, None] * dinv[None, :]

    h = jnp.zeros((_NP, _HD), jnp.float32)
    for t in range(_T):
        Ahat = norm_adj(dense_ref[2 * t])
        Ahats = norm_adj(dense_ref[2 * t + 1])
        xt = x_ref[t]
        phi_x = jax.nn.relu(mm(xt, w["phi_x_W"][...]) + w["phi_x_b"][...])
        phi_d = jax.nn.relu(mm(xt, w["phi_d_W"][...]) + w["phi_d_b"][...])
        cat1 = jnp.concatenate([phi_x, h], axis=1)
        enc = jax.nn.relu(mm(Ahat, mm(cat1, w["enc_W"][...])) + w["enc_b"][...])
        ms = mm(Ahat, mm(enc, w["Wms"][...])) + w["bms"][...]
        enc_mean, enc_std = ms[:, :_EX], _softplus(ms[:, _EX:])
        encd = jax.nn.relu(
            mm(Ahats, mm(phi_d, w["sup_enc_W"][...])) + w["sup_enc_b"][...])
        msd = mm(Ahats, mm(encd, w["Wmss"][...])) + w["bmss"][...]
        encd_mean, encd_std = msd[:, :_EX], _softplus(msd[:, _EX:])
        e_x = enc_mean + enc_std * epsx_ref[t]
        e_d = encd_mean + encd_std * epsd_ref[t]
        phi_e_x = jax.nn.relu(
            mm(e_x, w["phi_e_x_W"][...]) + w["phi_e_x_b"][...])
        ed = jnp.concatenate([e_x, e_d], axis=1)
        zi = _prelu(mm(ed, w["zin1_W"][...]) + w["zin1_b"][...],
                    w["zin_a1"][0, 0])
        zi = _prelu(mm(zi, w["zin2_W"][...]) + w["zin2_b"][...],
                    w["zin_a2"][0, 0])
        zo = _prelu(mm(ed, w["zout1_W"][...]) + w["zout1_b"][...],
                    w["zout_a1"][0, 0])
        zo = _prelu(mm(zo, w["zout2_W"][...]) + w["zout2_b"][...],
                    w["zout_a2"][0, 0])
        av = mm(zo, w["dec1_Wo"][...]) + w["dec1_b"][...]
        bv = mm(zi, w["dec1_Wi"][...])
        a_ref[t] = av
        b_ref[t] = bv
        a1 = w["dec_a1"][0, 0]
        cval = a1 * mm(av, w["dec2_W"][...]) + w["dec2_b"][...]
        dval = a1 * mm(bv, w["dec2_W"][...])
        c_ref[t] = cval
        d_ref[t] = dval
        inp = jnp.concatenate([phi_x, phi_e_x], axis=1)
        zr = jax.nn.sigmoid(
            mm(Ahat, mm(inp, w["Wxzr"][...]) + mm(h, w["Whzr"][...]))
            + w["bzr"][...])
        z_g, r_g = zr[:, :_HD], zr[:, _HD:]
        pre_h = mm(Ahat, mm(inp, w["rnn_xh_W"][...])
                   + mm(r_g * h, w["rnn_hh_W"][...])) + w["bh"][...]
        h = z_g * h + (1.0 - z_g) * jnp.tanh(pre_h)


_BI = 32


def _dec_body(a_ref, b_ref, c_ref, d_ref, w2p_ref,
              w3r_ref, b3_ref, a2_ref, out_ref):
    a = a_ref[0]
    b = b_ref[0]
    a2 = a2_ref[0, 0]
    r = jnp.maximum(a[:, None, :] + b[None, :, :], 0.0).reshape(
        _BI * _NP, _HD)
    m = jnp.dot(r, w2p_ref[...], preferred_element_type=jnp.float32)
    q = m.reshape(_BI, _NP, _HD) + c_ref[0][:, None, :] + d_ref[0][None, :, :]
    h2 = _prelu(q, a2).reshape(_BI * _NP, _HD)
    s = lax.dot_general(w3r_ref[...], h2, (((1,), (1,)), ((), ())),
                        preferred_element_type=jnp.float32)
    pre = s.reshape(_BI, _NP) + b3_ref[0, 0]
    out_ref[0] = jax.nn.sigmoid(pre)


def kernel(x, edge_index, edge_weight, edge_index_sup, edge_weight_sup,
           mask, A_scaler, truths, eps_x, eps_d, h0, params):
    p = params
    f32 = jnp.float32

    def flat(ei, ew):
        src = ei[:, 0, :].astype(jnp.int32)
        dst = ei[:, 1, :].astype(jnp.int32)
        return dst * _NP + src, ew.astype(f32)

    fi_m, fw_m = flat(edge_index, edge_weight)
    fi_s, fw_s = flat(edge_index_sup, edge_weight_sup)
    fi = jnp.stack([fi_m[0], fi_s[0], fi_m[1], fi_s[1]])
    fw = jnp.stack([fw_m[0], fw_s[0], fw_m[1], fw_s[1]])
    pad = _EPW_PAD - _EPW
    fi = jnp.pad(fi.reshape(_NSETS, _NS, _EPW), ((0, 0), (0, 0), (0, pad)))
    fw = jnp.pad(fw.reshape(_NSETS, _NS, _EPW), ((0, 0), (0, 0), (0, pad)))
    fi = fi.reshape(_NSETS, _NS, 8, 128)
    fw = fw.reshape(_NSETS, _NS, 8, 128)
    zeros_chunk = jnp.zeros((_CHUNK,), f32)

    dense = _densify_sc(fi, fw, zeros_chunk).reshape(_NSETS, _NP, _NP)

    padn = _NP - _N
    x_p = jnp.pad(x, ((0, 0), (0, padn), (0, 0)))
    epsx_p = jnp.pad(eps_x, ((0, 0), (0, padn), (0, 0)))
    epsd_p = jnp.pad(eps_d, ((0, 0), (0, padn), (0, 0)))

    def row(v):
        return v.reshape(1, -1)

    wvals = {
        "phi_x_W": p["phi_x_W"], "phi_x_b": row(p["phi_x_b"]),
        "phi_d_W": p["phi_d_W"], "phi_d_b": row(p["phi_d_b"]),
        "phi_e_x_W": p["phi_e_x_W"], "phi_e_x_b": row(p["phi_e_x_b"]),
        "enc_W": p["enc_W"], "enc_b": row(p["enc_b"]),
        "Wms": jnp.concatenate([p["enc_mean_W"], p["enc_std_W"]], 1),
        "bms": row(jnp.concatenate([p["enc_mean_b"], p["enc_std_b"]])),
        "sup_enc_W": p["sup_enc_W"], "sup_enc_b": row(p["sup_enc_b"]),
        "Wmss": jnp.concatenate([p["sup_enc_mean_W"], p["sup_enc_std_W"]], 1),
        "bmss": row(jnp.concatenate([p["sup_enc_mean_b"],
                                     p["sup_enc_std_b"]])),
        "zin1_W": p["zin1_W"], "zin1_b": row(p["zin1_b"]),
        "zin2_W": p["zin2_W"], "zin2_b": row(p["zin2_b"]),
        "zout1_W": p["zout1_W"], "zout1_b": row(p["zout1_b"]),
        "zout2_W": p["zout2_W"], "zout2_b": row(p["zout2_b"]),
        "dec1_Wo": p["dec1_W"][:64], "dec1_Wi": p["dec1_W"][64:],
        "dec1_b": row(p["dec1_b"]),
        "Wxzr": jnp.concatenate([p["rnn_xz_W"], p["rnn_xr_W"]], 1),
        "Whzr": jnp.concatenate([p["rnn_hz_W"], p["rnn_hr_W"]], 1),
        "bzr": row(jnp.concatenate([p["rnn_xz_b"] + p["rnn_hz_b"],
                                    p["rnn_xr_b"] + p["rnn_hr_b"]])),
        "rnn_xh_W": p["rnn_xh_W"], "rnn_hh_W": p["rnn_hh_W"],
        "bh": row(p["rnn_xh_b"] + p["rnn_hh_b"]),
        "zin_a1": p["zin_a1"].reshape(1, 1),
        "zin_a2": p["zin_a2"].reshape(1, 1),
        "zout_a1": p["zout_a1"].reshape(1, 1),
        "zout_a2": p["zout_a2"].reshape(1, 1),
        "dec2_W": p["dec2_W"], "dec2_b": row(p["dec2_b"]),
        "dec_a1": p["dec_a1"].reshape(1, 1),
        "dec3_W": p["dec3_W"], "dec3_Wr": p["dec3_W"].reshape(1, _HD),
    }
    prep_in = [dense, x_p, epsx_p, epsd_p] + [wvals[n] for n in _PREP_W]

    A, B, C, D = pl.pallas_call(
        _prep_body,
        out_shape=[
            jax.ShapeDtypeStruct((_T, _NP, _HD), f32),
            jax.ShapeDtypeStruct((_T, _NP, _HD), f32),
            jax.ShapeDtypeStruct((_T, _NP, _HD), f32),
            jax.ShapeDtypeStruct((_T, _NP, _HD), f32),
        ],
    )(*prep_in)

    w2p = (1.0 - p["dec_a1"]) * p["dec2_W"]
    out = pl.pallas_call(
        _dec_body,
        grid=(_T, _NP // _BI),
        in_specs=[
            pl.BlockSpec((1, _BI, _HD), lambda t, i: (t, i, 0)),
            pl.BlockSpec((1, _NP, _HD), lambda t, i: (t, 0, 0)),
            pl.BlockSpec((1, _BI, _HD), lambda t, i: (t, i, 0)),
            pl.BlockSpec((1, _NP, _HD), lambda t, i: (t, 0, 0)),
            pl.BlockSpec((_HD, _HD), lambda t, i: (0, 0)),
            pl.BlockSpec((1, _HD), lambda t, i: (0, 0)),
            pl.BlockSpec((1, 1), lambda t, i: (0, 0)),
            pl.BlockSpec((1, 1), lambda t, i: (0, 0)),
        ],
        out_specs=pl.BlockSpec((1, _BI, _NP), lambda t, i: (t, i, 0)),
        out_shape=jax.ShapeDtypeStruct((_T, _NP, _NP), f32),
        compiler_params=pltpu.CompilerParams(
            dimension_semantics=("parallel", "parallel")),
    )(A, B, C, D, w2p, p["dec3_W"].reshape(1, _HD),
      p["dec3_b"].reshape(1, 1), p["dec_a2"].reshape(1, 1))

    return out[:, :_N, :_N]


# speedup vs baseline: 17.2029x; 1.0253x over previous
import functools

import jax
import jax.numpy as jnp
from jax import lax
from jax.experimental import pallas as pl
from jax.experimental.pallas import tpu as pltpu
from jax.experimental.pallas import tpu_sc as plsc

_N = 500
_T = 2
_HD = 256
_EX = 128
_E = 16000
_NP = 512
_P = _NP * _NP
_NSETS = 4
_NC = 2
_NS = 16
_EPW = _E // _NS
_EPW_PAD = 1024
_CHUNK = _P // _NS


def _densify_sc(flat_idx, flat_w, zeros_chunk):
    mesh = plsc.VectorSubcoreMesh(
        core_axis_name="c", subcore_axis_name="s", num_cores=_NC,
        num_subcores=_NS)

    @functools.partial(
        pl.kernel,
        out_type=jax.ShapeDtypeStruct((_NSETS, _P), jnp.float32),
        mesh=mesh,
        scratch_types=[
            pltpu.VMEM((8, 128), jnp.int32),
            pltpu.VMEM((8, 128), jnp.float32),
            pltpu.VMEM_SHARED((_P,), jnp.float32),
            pltpu.VMEM_SHARED((_P,), jnp.float32),
        ],
    )
    def body(idx_hbm, w_hbm, z_hbm, out_hbm, idx_v, w_v, sh0, sh1):
        c = lax.axis_index("c")
        s = lax.axis_index("s")
        for j, sh in enumerate((sh0, sh1)):
            set_id = c * 2 + j
            pltpu.sync_copy(z_hbm, sh.at[pl.ds(s * _CHUNK, _CHUNK)])
            pltpu.sync_copy(idx_hbm.at[set_id, s], idx_v)
            pltpu.sync_copy(w_hbm.at[set_id, s], w_v)
            plsc.subcore_barrier()
            for k in range(8):
                pltpu.sync_copy(w_v.at[k], sh.at[idx_v.at[k]], add=True)
            plsc.subcore_barrier()
            pltpu.sync_copy(
                sh.at[pl.ds(s * _CHUNK, _CHUNK)],
                out_hbm.at[set_id, pl.ds(s * _CHUNK, _CHUNK)])

    return body(flat_idx, flat_w, zeros_chunk)


def _softplus(x):
    return jnp.log1p(jnp.exp(-jnp.abs(x))) + jnp.maximum(x, 0.0)


def _prelu(x, a):
    return jnp.where(x >= 0, x, a * x)


_PREP_W = [
    "phi_x_W", "phi_x_b", "phi_d_W", "phi_d_b", "phi_e_x_W", "phi_e_x_b",
    "enc_W", "enc_b", "Wms", "bms", "sup_enc_W", "sup_enc_b", "Wmss", "bmss",
    "zin1_W", "zin1_b", "zin2_W", "zin2_b", "zout1_W", "zout1_b",
    "zout2_W", "zout2_b", "dec1_Wo", "dec1_Wi", "dec1_b",
    "Wxzr", "Whzr", "bzr", "rnn_xh_W", "rnn_hh_W", "bh",
    "zin_a1", "zin_a2", "zout_a1", "zout_a2",
    "dec2_W", "dec2_b", "dec_a1", "dec3_W", "dec3_Wr",
]


def _prep_body(*refs):
    dense_ref, x_ref, epsx_ref, epsd_ref = refs[:4]
    rest = refs[4:]
    w = dict(zip(_PREP_W, rest[:len(_PREP_W)]))
    a_ref, b_ref, c_ref, d_ref = rest[len(_PREP_W):]

    def mm(a, b):
        return jnp.dot(a, b, preferred_element_type=jnp.float32)

    rows = lax.broadcasted_iota(jnp.int32, (_NP, _NP), 0)
    cols = lax.broadcasted_iota(jnp.int32, (_NP, _NP), 1)
    diag = (rows == cols).astype(jnp.float32)

    def norm_adj(S):
        deg = jnp.sum(S, axis=1) + 1.0
        dinv = lax.rsqrt(deg)
        return (S + diag) * dinv[:, None] * dinv[None, :]

    h = jnp.zeros((_NP, _HD), jnp.float32)
    for t in range(_T):
        Ahat = norm_adj(dense_ref[2 * t])
        Ahats = norm_adj(dense_ref[2 * t + 1])
        xt = x_ref[t]
        phi_x = jax.nn.relu(mm(xt, w["phi_x_W"][...]) + w["phi_x_b"][...])
        phi_d = jax.nn.relu(mm(xt, w["phi_d_W"][...]) + w["phi_d_b"][...])
        cat1 = jnp.concatenate([phi_x, h], axis=1)
        enc = jax.nn.relu(mm(Ahat, mm(cat1, w["enc_W"][...])) + w["enc_b"][...])
        ms = mm(Ahat, mm(enc, w["Wms"][...])) + w["bms"][...]
        enc_mean, enc_std = ms[:, :_EX], _softplus(ms[:, _EX:])
        encd = jax.nn.relu(
            mm(Ahats, mm(phi_d, w["sup_enc_W"][...])) + w["sup_enc_b"][...])
        msd = mm(Ahats, mm(encd, w["Wmss"][...])) + w["bmss"][...]
        encd_mean, encd_std = msd[:, :_EX], _softplus(msd[:, _EX:])
        e_x = enc_mean + enc_std * epsx_ref[t]
        e_d = encd_mean + encd_std * epsd_ref[t]
        phi_e_x = jax.nn.relu(
            mm(e_x, w["phi_e_x_W"][...]) + w["phi_e_x_b"][...])
        ed = jnp.concatenate([e_x, e_d], axis=1)
        zi = _prelu(mm(ed, w["zin1_W"][...]) + w["zin1_b"][...],
                    w["zin_a1"][0, 0])
        zi = _prelu(mm(zi, w["zin2_W"][...]) + w["zin2_b"][...],
                    w["zin_a2"][0, 0])
        zo = _prelu(mm(ed, w["zout1_W"][...]) + w["zout1_b"][...],
                    w["zout_a1"][0, 0])
        zo = _prelu(mm(zo, w["zout2_W"][...]) + w["zout2_b"][...],
                    w["zout_a2"][0, 0])
        av = mm(zo, w["dec1_Wo"][...]) + w["dec1_b"][...]
        bv = mm(zi, w["dec1_Wi"][...])
        a_ref[t] = av
        b_ref[t] = bv
        a1 = w["dec_a1"][0, 0]
        cval = a1 * mm(av, w["dec2_W"][...]) + w["dec2_b"][...]
        dval = a1 * mm(bv, w["dec2_W"][...])
        c_ref[t] = cval
        d_ref[t] = dval
        inp = jnp.concatenate([phi_x, phi_e_x], axis=1)
        zr = jax.nn.sigmoid(
            mm(Ahat, mm(inp, w["Wxzr"][...]) + mm(h, w["Whzr"][...]))
            + w["bzr"][...])
        z_g, r_g = zr[:, :_HD], zr[:, _HD:]
        pre_h = mm(Ahat, mm(inp, w["rnn_xh_W"][...])
                   + mm(r_g * h, w["rnn_hh_W"][...])) + w["bh"][...]
        h = z_g * h + (1.0 - z_g) * jnp.tanh(pre_h)


_BI = 64


def _dec_body(a_ref, b_ref, c_ref, d_ref, w2p_ref,
              w3r_ref, b3_ref, a2_ref, out_ref):
    a = a_ref[0]
    b = b_ref[0]
    a2 = a2_ref[0, 0]
    r = jnp.maximum(a[:, None, :] + b[None, :, :], 0.0).reshape(
        _BI * _NP, _HD)
    m = jnp.dot(r, w2p_ref[...], preferred_element_type=jnp.float32)
    q = m.reshape(_BI, _NP, _HD) + c_ref[0][:, None, :] + d_ref[0][None, :, :]
    h2 = _prelu(q, a2).reshape(_BI * _NP, _HD)
    s = lax.dot_general(w3r_ref[...], h2, (((1,), (1,)), ((), ())),
                        preferred_element_type=jnp.float32)
    pre = s.reshape(_BI, _NP) + b3_ref[0, 0]
    out_ref[0] = jax.nn.sigmoid(pre)


def kernel(x, edge_index, edge_weight, edge_index_sup, edge_weight_sup,
           mask, A_scaler, truths, eps_x, eps_d, h0, params):
    p = params
    f32 = jnp.float32

    def flat(ei, ew):
        src = ei[:, 0, :].astype(jnp.int32)
        dst = ei[:, 1, :].astype(jnp.int32)
        return dst * _NP + src, ew.astype(f32)

    fi_m, fw_m = flat(edge_index, edge_weight)
    fi_s, fw_s = flat(edge_index_sup, edge_weight_sup)
    fi = jnp.stack([fi_m[0], fi_s[0], fi_m[1], fi_s[1]])
    fw = jnp.stack([fw_m[0], fw_s[0], fw_m[1], fw_s[1]])
    pad = _EPW_PAD - _EPW
    fi = jnp.pad(fi.reshape(_NSETS, _NS, _EPW), ((0, 0), (0, 0), (0, pad)))
    fw = jnp.pad(fw.reshape(_NSETS, _NS, _EPW), ((0, 0), (0, 0), (0, pad)))
    fi = fi.reshape(_NSETS, _NS, 8, 128)
    fw = fw.reshape(_NSETS, _NS, 8, 128)
    zeros_chunk = jnp.zeros((_CHUNK,), f32)

    dense = _densify_sc(fi, fw, zeros_chunk).reshape(_NSETS, _NP, _NP)

    padn = _NP - _N
    x_p = jnp.pad(x, ((0, 0), (0, padn), (0, 0)))
    epsx_p = jnp.pad(eps_x, ((0, 0), (0, padn), (0, 0)))
    epsd_p = jnp.pad(eps_d, ((0, 0), (0, padn), (0, 0)))

    def row(v):
        return v.reshape(1, -1)

    wvals = {
        "phi_x_W": p["phi_x_W"], "phi_x_b": row(p["phi_x_b"]),
        "phi_d_W": p["phi_d_W"], "phi_d_b": row(p["phi_d_b"]),
        "phi_e_x_W": p["phi_e_x_W"], "phi_e_x_b": row(p["phi_e_x_b"]),
        "enc_W": p["enc_W"], "enc_b": row(p["enc_b"]),
        "Wms": jnp.concatenate([p["enc_mean_W"], p["enc_std_W"]], 1),
        "bms": row(jnp.concatenate([p["enc_mean_b"], p["enc_std_b"]])),
        "sup_enc_W": p["sup_enc_W"], "sup_enc_b": row(p["sup_enc_b"]),
        "Wmss": jnp.concatenate([p["sup_enc_mean_W"], p["sup_enc_std_W"]], 1),
        "bmss": row(jnp.concatenate([p["sup_enc_mean_b"],
                                     p["sup_enc_std_b"]])),
        "zin1_W": p["zin1_W"], "zin1_b": row(p["zin1_b"]),
        "zin2_W": p["zin2_W"], "zin2_b": row(p["zin2_b"]),
        "zout1_W": p["zout1_W"], "zout1_b": row(p["zout1_b"]),
        "zout2_W": p["zout2_W"], "zout2_b": row(p["zout2_b"]),
        "dec1_Wo": p["dec1_W"][:64], "dec1_Wi": p["dec1_W"][64:],
        "dec1_b": row(p["dec1_b"]),
        "Wxzr": jnp.concatenate([p["rnn_xz_W"], p["rnn_xr_W"]], 1),
        "Whzr": jnp.concatenate([p["rnn_hz_W"], p["rnn_hr_W"]], 1),
        "bzr": row(jnp.concatenate([p["rnn_xz_b"] + p["rnn_hz_b"],
                                    p["rnn_xr_b"] + p["rnn_hr_b"]])),
        "rnn_xh_W": p["rnn_xh_W"], "rnn_hh_W": p["rnn_hh_W"],
        "bh": row(p["rnn_xh_b"] + p["rnn_hh_b"]),
        "zin_a1": p["zin_a1"].reshape(1, 1),
        "zin_a2": p["zin_a2"].reshape(1, 1),
        "zout_a1": p["zout_a1"].reshape(1, 1),
        "zout_a2": p["zout_a2"].reshape(1, 1),
        "dec2_W": p["dec2_W"], "dec2_b": row(p["dec2_b"]),
        "dec_a1": p["dec_a1"].reshape(1, 1),
        "dec3_W": p["dec3_W"], "dec3_Wr": p["dec3_W"].reshape(1, _HD),
    }
    prep_in = [dense, x_p, epsx_p, epsd_p] + [wvals[n] for n in _PREP_W]

    A, B, C, D = pl.pallas_call(
        _prep_body,
        out_shape=[
            jax.ShapeDtypeStruct((_T, _NP, _HD), f32),
            jax.ShapeDtypeStruct((_T, _NP, _HD), f32),
            jax.ShapeDtypeStruct((_T, _NP, _HD), f32),
            jax.ShapeDtypeStruct((_T, _NP, _HD), f32),
        ],
    )(*prep_in)

    w2p = (1.0 - p["dec_a1"]) * p["dec2_W"]
    out = pl.pallas_call(
        _dec_body,
        grid=(_T, _NP // _BI),
        in_specs=[
            pl.BlockSpec((1, _BI, _HD), lambda t, i: (t, i, 0)),
            pl.BlockSpec((1, _NP, _HD), lambda t, i: (t, 0, 0)),
            pl.BlockSpec((1, _BI, _HD), lambda t, i: (t, i, 0)),
            pl.BlockSpec((1, _NP, _HD), lambda t, i: (t, 0, 0)),
            pl.BlockSpec((_HD, _HD), lambda t, i: (0, 0)),
            pl.BlockSpec((1, _HD), lambda t, i: (0, 0)),
            pl.BlockSpec((1, 1), lambda t, i: (0, 0)),
            pl.BlockSpec((1, 1), lambda t, i: (0, 0)),
        ],
        out_specs=pl.BlockSpec((1, _BI, _NP), lambda t, i: (t, i, 0)),
        out_shape=jax.ShapeDtypeStruct((_T, _NP, _NP), f32),
        compiler_params=pltpu.CompilerParams(
            dimension_semantics=("parallel", "parallel")),
    )(A, B, C, D, w2p, p["dec3_W"].reshape(1, _HD),
      p["dec3_b"].reshape(1, 1), p["dec_a2"].reshape(1, 1))

    return out[:, :_N, :_N]
